# baseline (speedup 1.0000x reference)
import functools

import jax
import jax.numpy as jnp
from jax import lax
from jax.experimental import pallas as pl
from jax.experimental.pallas import tpu as pltpu


def _titans_body(H, D, L, NC, x_ref, wq_ref, wk_ref, wv_ref, wp_ref,
                 ab_ref, state_ref, out_ref, stateout_ref,
                 s_ref, qt_ref, kt_ref, vt_ref, yt_ref):
    c = pl.program_id(1)
    f32 = jnp.float32
    c11 = (((1,), (1,)), ((), ()))
    c00 = (((0,), (0,)), ((), ()))

    @pl.when(c == 0)
    def _():
        s_ref[...] = state_ref[0]

    xb = x_ref[0]
    qt_ref[...] = lax.dot_general(wq_ref[...], xb, c11, preferred_element_type=f32)
    kt_ref[...] = lax.dot_general(wk_ref[...], xb, c11, preferred_element_type=f32)
    vt_ref[...] = lax.dot_general(wv_ref[...], xb, c11, preferred_element_type=f32)

    ab = 0.5 * jax.nn.sigmoid(ab_ref[...])

    row = lax.broadcasted_iota(jnp.int32, (L, L), 0)
    col = lax.broadcasted_iota(jnp.int32, (L, L), 1)
    umask = row < col

    GRP = 8
    outs = []
    for h0 in range(0, H, GRP):
        hs = range(h0, min(h0 + GRP, H))
        Qt, Kt, Vt, S, al, be = {}, {}, {}, {}, {}, {}
        for h in hs:
            sl = slice(h * D, (h + 1) * D)
            Qt[h] = qt_ref[sl, :]
            Vt[h] = vt_ref[sl, :]
            K = kt_ref[sl, :]
            nrm = jnp.sqrt(jnp.sum(K * K, axis=0, keepdims=True))
            Kt[h] = K / jnp.maximum(nrm, 1e-12)
            al[h] = ab[0:1, h:h + 1]
            be[h] = ab[1:2, h:h + 1]
            S[h] = s_ref[h]
        Nt, Ut = {}, {}
        for h in hs:
            At = lax.dot_general(Kt[h], Kt[h], c00, preferred_element_type=f32)
            Nt[h] = jnp.where(umask, al[h] * At, 0.0)
        for h in hs:
            Rt = be[h] * Vt[h] - al[h] * jnp.dot(S[h], Kt[h],
                                                 preferred_element_type=f32)
            Ut[h] = Rt - jnp.dot(Rt, Nt[h], preferred_element_type=f32)
        P = {h: Nt[h].astype(jnp.bfloat16) for h in hs}
        p = 2
        while p <= L // 2:
            for h in hs:
                P[h] = jnp.dot(P[h], P[h],
                               preferred_element_type=f32).astype(jnp.bfloat16)
            for h in hs:
                Ut[h] = Ut[h] + jnp.dot(Ut[h], P[h].astype(f32),
                                        preferred_element_type=f32)
            p *= 2
        for h in hs:
            G = lax.dot_general(Kt[h], Qt[h], c00, preferred_element_type=f32)
            Yt = jnp.dot(S[h], Qt[h], preferred_element_type=f32) + \
                jnp.dot(Ut[h], jnp.where(umask, G, 0.0),
                        preferred_element_type=f32)
            yt_ref[h * D:(h + 1) * D, :] = Yt
        for h in hs:
            s_ref[h] = S[h] + lax.dot_general(Ut[h], Kt[h], c11,
                                              preferred_element_type=f32)
        gsl = slice(h0 * D, (h0 + len(hs)) * D)
        outs.append(lax.dot_general(yt_ref[gsl, :], wp_ref[:, gsl],
                                    (((0,), (1,)), ((), ())),
                                    preferred_element_type=f32))

    acc = outs[0]
    for og in outs[1:]:
        acc = acc + og
    out_ref[0] = acc

    @pl.when(c == NC - 1)
    def _():
        stateout_ref[0] = s_ref[...]


def kernel(x, Wq, Wk, Wv, Wproj, alpha_raw, beta_raw, state):
    B, T, C = x.shape
    H = alpha_raw.shape[1]
    D = C // H
    L = 256 if T % 256 == 0 else (128 if T % 128 == 0 else T)
    NC = T // L
    ab = jnp.stack([alpha_raw.reshape(H), beta_raw.reshape(H)])

    body = functools.partial(_titans_body, H, D, L, NC)
    wspec = pl.BlockSpec((C, C), lambda b, c: (0, 0))
    out, state_f = pl.pallas_call(
        body,
        grid=(B, NC),
        in_specs=[
            pl.BlockSpec((1, L, C), lambda b, c: (b, c, 0)),
            wspec, wspec, wspec, wspec,
            pl.BlockSpec((2, H), lambda b, c: (0, 0)),
            pl.BlockSpec((1, H, D, D), lambda b, c: (b, 0, 0, 0)),
        ],
        out_specs=[
            pl.BlockSpec((1, L, C), lambda b, c: (b, c, 0)),
            pl.BlockSpec((1, H, D, D), lambda b, c: (b, 0, 0, 0)),
        ],
        out_shape=[
            jax.ShapeDtypeStruct((B, T, C), jnp.float32),
            jax.ShapeDtypeStruct((B, H, D, D), jnp.float32),
        ],
        scratch_shapes=[
            pltpu.VMEM((H, D, D), jnp.float32),
            pltpu.VMEM((C, L), jnp.float32),
            pltpu.VMEM((C, L), jnp.float32),
            pltpu.VMEM((C, L), jnp.float32),
            pltpu.VMEM((C, L), jnp.float32),
        ],
        compiler_params=pltpu.CompilerParams(
            dimension_semantics=("parallel", "arbitrary"),
            vmem_limit_bytes=56 * 1024 * 1024,
        ),
        name="titans_l2_chunked",
    )(x, Wq, Wk, Wv, Wproj, ab, state)
    return out, state_f


# speedup vs baseline: 1.0324x; 1.0324x over previous
import functools

import jax
import jax.numpy as jnp
from jax import lax
from jax.experimental import pallas as pl
from jax.experimental.pallas import tpu as pltpu


def _titans_body(H, D, L, NC, BB, GRP, x_ref, wq_ref, wk_ref, wv_ref, wp_ref,
                 ab_ref, state_ref, out_ref, stateout_ref,
                 s_ref, qt_ref, kt_ref, vt_ref, yt_ref):
    c = pl.program_id(1)
    f32 = jnp.float32
    c11 = (((1,), (1,)), ((), ()))
    c00 = (((0,), (0,)), ((), ()))

    @pl.when(c == 0)
    def _():
        s_ref[...] = state_ref[...]

    for bi in range(BB):
        xb = x_ref[bi]
        qt_ref[bi] = lax.dot_general(wq_ref[...], xb, c11,
                                     preferred_element_type=f32)
        kt_ref[bi] = lax.dot_general(wk_ref[...], xb, c11,
                                     preferred_element_type=f32)
        vt_ref[bi] = lax.dot_general(wv_ref[...], xb, c11,
                                     preferred_element_type=f32)

    ab = 0.5 * jax.nn.sigmoid(ab_ref[...])

    row = lax.broadcasted_iota(jnp.int32, (L, L), 0)
    col = lax.broadcasted_iota(jnp.int32, (L, L), 1)
    umask = row < col

    pairs_all = [(bi, h) for bi in range(BB) for h in range(H)]
    for g0 in range(0, len(pairs_all), GRP):
        ps = pairs_all[g0:g0 + GRP]
        Qt, Kt, Vt, S, al, be = {}, {}, {}, {}, {}, {}
        for bh in ps:
            bi, h = bh
            sl = slice(h * D, (h + 1) * D)
            Qt[bh] = qt_ref[bi, sl, :]
            Vt[bh] = vt_ref[bi, sl, :]
            K = kt_ref[bi, sl, :]
            nrm = jnp.sqrt(jnp.sum(K * K, axis=0, keepdims=True))
            Kt[bh] = K / jnp.maximum(nrm, 1e-12)
            al[bh] = ab[0:1, h:h + 1]
            be[bh] = ab[1:2, h:h + 1]
            S[bh] = s_ref[bi, h]
        Nt, Ut = {}, {}
        for bh in ps:
            At = lax.dot_general(Kt[bh], Kt[bh], c00, preferred_element_type=f32)
            Nt[bh] = jnp.where(umask, al[bh] * At, 0.0)
        for bh in ps:
            Rt = be[bh] * Vt[bh] - al[bh] * jnp.dot(S[bh], Kt[bh],
                                                    preferred_element_type=f32)
            Ut[bh] = Rt - jnp.dot(Rt, Nt[bh], preferred_element_type=f32)
        P = dict(Nt)
        p = 2
        while p <= L // 2:
            for bh in ps:
                P[bh] = jnp.dot(P[bh], P[bh], preferred_element_type=f32)
            for bh in ps:
                Ut[bh] = Ut[bh] + jnp.dot(Ut[bh], P[bh],
                                          preferred_element_type=f32)
            p *= 2
        for bh in ps:
            bi, h = bh
            G = lax.dot_general(Kt[bh], Qt[bh], c00, preferred_element_type=f32)
            Yt = jnp.dot(S[bh], Qt[bh], preferred_element_type=f32) + \
                jnp.dot(Ut[bh], jnp.where(umask, G, 0.0),
                        preferred_element_type=f32)
            yt_ref[bi, h * D:(h + 1) * D, :] = Yt
        for bh in ps:
            bi, h = bh
            s_ref[bi, h] = S[bh] + lax.dot_general(Ut[bh], Kt[bh], c11,
                                                   preferred_element_type=f32)

    for bi in range(BB):
        out_ref[bi] = lax.dot_general(yt_ref[bi], wp_ref[...],
                                      (((0,), (1,)), ((), ())),
                                      preferred_element_type=f32)

    @pl.when(c == NC - 1)
    def _():
        stateout_ref[...] = s_ref[...]


def kernel(x, Wq, Wk, Wv, Wproj, alpha_raw, beta_raw, state):
    B, T, C = x.shape
    H = alpha_raw.shape[1]
    D = C // H
    L = 256 if T % 256 == 0 else (128 if T % 128 == 0 else T)
    NC = T // L
    BB = 2 if B % 2 == 0 else 1
    GRP = 16
    ab = jnp.stack([alpha_raw.reshape(H), beta_raw.reshape(H)])

    body = functools.partial(_titans_body, H, D, L, NC, BB, GRP)
    wspec = pl.BlockSpec((C, C), lambda b, c: (0, 0))
    out, state_f = pl.pallas_call(
        body,
        grid=(B // BB, NC),
        in_specs=[
            pl.BlockSpec((BB, L, C), lambda b, c: (b, c, 0)),
            wspec, wspec, wspec, wspec,
            pl.BlockSpec((2, H), lambda b, c: (0, 0)),
            pl.BlockSpec((BB, H, D, D), lambda b, c: (b, 0, 0, 0)),
        ],
        out_specs=[
            pl.BlockSpec((BB, L, C), lambda b, c: (b, c, 0)),
            pl.BlockSpec((BB, H, D, D), lambda b, c: (b, 0, 0, 0)),
        ],
        out_shape=[
            jax.ShapeDtypeStruct((B, T, C), jnp.float32),
            jax.ShapeDtypeStruct((B, H, D, D), jnp.float32),
        ],
        scratch_shapes=[
            pltpu.VMEM((BB, H, D, D), jnp.float32),
            pltpu.VMEM((BB, C, L), jnp.float32),
            pltpu.VMEM((BB, C, L), jnp.float32),
            pltpu.VMEM((BB, C, L), jnp.float32),
            pltpu.VMEM((BB, C, L), jnp.float32),
        ],
        compiler_params=pltpu.CompilerParams(
            dimension_semantics=("parallel", "arbitrary"),
            vmem_limit_bytes=56 * 1024 * 1024,
        ),
        name="titans_l2_chunked",
    )(x, Wq, Wk, Wv, Wproj, ab, state)
    return out, state_f


# speedup vs baseline: 1.0396x; 1.0070x over previous
import functools

import jax
import jax.numpy as jnp
from jax import lax
from jax.experimental import pallas as pl
from jax.experimental.pallas import tpu as pltpu


def _titans_body(H, D, L, NC, BB, GRP, x_ref, wq_ref, wk_ref, wv_ref, wp_ref,
                 ab_ref, state_ref, out_ref, stateout_ref,
                 s_ref, qt_ref, kt_ref, vt_ref, yt_ref):
    c = pl.program_id(1)
    f32 = jnp.float32
    c11 = (((1,), (1,)), ((), ()))
    c00 = (((0,), (0,)), ((), ()))

    @pl.when(c == 0)
    def _():
        s_ref[...] = state_ref[...]

    bf16 = jnp.bfloat16
    wqb = wq_ref[...].astype(bf16)
    wkb = wk_ref[...].astype(bf16)
    wvb = wv_ref[...].astype(bf16)
    for bi in range(BB):
        xbb = x_ref[bi].astype(bf16)
        qt_ref[bi] = lax.dot_general(wqb, xbb, c11,
                                     preferred_element_type=f32)
        kt_ref[bi] = lax.dot_general(wkb, xbb, c11,
                                     preferred_element_type=f32)
        vt_ref[bi] = lax.dot_general(wvb, xbb, c11,
                                     preferred_element_type=f32)

    ab = 0.5 * jax.nn.sigmoid(ab_ref[...])

    row = lax.broadcasted_iota(jnp.int32, (L, L), 0)
    col = lax.broadcasted_iota(jnp.int32, (L, L), 1)
    umask = row < col

    pairs_all = [(bi, h) for bi in range(BB) for h in range(H)]
    for g0 in range(0, len(pairs_all), GRP):
        ps = pairs_all[g0:g0 + GRP]
        Qt, Kt, Vt, S, al, be = {}, {}, {}, {}, {}, {}
        for bh in ps:
            bi, h = bh
            sl = slice(h * D, (h + 1) * D)
            Qt[bh] = qt_ref[bi, sl, :]
            Vt[bh] = vt_ref[bi, sl, :]
            K = kt_ref[bi, sl, :]
            nrm = jnp.sqrt(jnp.sum(K * K, axis=0, keepdims=True))
            Kt[bh] = K / jnp.maximum(nrm, 1e-12)
            al[bh] = ab[0:1, h:h + 1]
            be[bh] = ab[1:2, h:h + 1]
            S[bh] = s_ref[bi, h]
        Nt, Ut = {}, {}
        for bh in ps:
            At = lax.dot_general(Kt[bh], Kt[bh], c00, preferred_element_type=f32)
            Nt[bh] = jnp.where(umask, al[bh] * At, 0.0)
        for bh in ps:
            Rt = be[bh] * Vt[bh] - al[bh] * jnp.dot(S[bh], Kt[bh],
                                                    preferred_element_type=f32)
            Ut[bh] = Rt - jnp.dot(Rt, Nt[bh], preferred_element_type=f32)
        P = dict(Nt)
        p = 2
        while p <= L // 2:
            for bh in ps:
                P[bh] = jnp.dot(P[bh], P[bh], preferred_element_type=f32)
            for bh in ps:
                Ut[bh] = Ut[bh] + jnp.dot(Ut[bh], P[bh],
                                          preferred_element_type=f32)
            p *= 2
        for bh in ps:
            bi, h = bh
            G = lax.dot_general(Kt[bh], Qt[bh], c00, preferred_element_type=f32)
            Yt = jnp.dot(S[bh], Qt[bh], preferred_element_type=f32) + \
                jnp.dot(Ut[bh], jnp.where(umask, G, 0.0),
                        preferred_element_type=f32)
            yt_ref[bi, h * D:(h + 1) * D, :] = Yt
        for bh in ps:
            bi, h = bh
            s_ref[bi, h] = S[bh] + lax.dot_general(Ut[bh], Kt[bh], c11,
                                                   preferred_element_type=f32)

    for bi in range(BB):
        out_ref[bi] = lax.dot_general(yt_ref[bi], wp_ref[...],
                                      (((0,), (1,)), ((), ())),
                                      preferred_element_type=f32)

    @pl.when(c == NC - 1)
    def _():
        stateout_ref[...] = s_ref[...]


def kernel(x, Wq, Wk, Wv, Wproj, alpha_raw, beta_raw, state):
    B, T, C = x.shape
    H = alpha_raw.shape[1]
    D = C // H
    L = 256 if T % 256 == 0 else (128 if T % 128 == 0 else T)
    NC = T // L
    BB = 2 if B % 2 == 0 else 1
    GRP = 16
    ab = jnp.stack([alpha_raw.reshape(H), beta_raw.reshape(H)])

    body = functools.partial(_titans_body, H, D, L, NC, BB, GRP)
    wspec = pl.BlockSpec((C, C), lambda b, c: (0, 0))
    out, state_f = pl.pallas_call(
        body,
        grid=(B // BB, NC),
        in_specs=[
            pl.BlockSpec((BB, L, C), lambda b, c: (b, c, 0)),
            wspec, wspec, wspec, wspec,
            pl.BlockSpec((2, H), lambda b, c: (0, 0)),
            pl.BlockSpec((BB, H, D, D), lambda b, c: (b, 0, 0, 0)),
        ],
        out_specs=[
            pl.BlockSpec((BB, L, C), lambda b, c: (b, c, 0)),
            pl.BlockSpec((BB, H, D, D), lambda b, c: (b, 0, 0, 0)),
        ],
        out_shape=[
            jax.ShapeDtypeStruct((B, T, C), jnp.float32),
            jax.ShapeDtypeStruct((B, H, D, D), jnp.float32),
        ],
        scratch_shapes=[
            pltpu.VMEM((BB, H, D, D), jnp.float32),
            pltpu.VMEM((BB, C, L), jnp.float32),
            pltpu.VMEM((BB, C, L), jnp.float32),
            pltpu.VMEM((BB, C, L), jnp.float32),
            pltpu.VMEM((BB, C, L), jnp.float32),
        ],
        compiler_params=pltpu.CompilerParams(
            dimension_semantics=("parallel", "arbitrary"),
            vmem_limit_bytes=56 * 1024 * 1024,
        ),
        name="titans_l2_chunked",
    )(x, Wq, Wk, Wv, Wproj, ab, state)
    return out, state_f
